# Initial kernel scaffold; baseline (speedup 1.0000x reference)
import jax
import jax.numpy as jnp
from jax.experimental import pallas as pl


def kernel(position, embed_table):
    raise NotImplementedError("write your pallas kernel here")


# speedup vs baseline: 1.0725x; 1.0725x over previous
import functools

import jax
import jax.numpy as jnp
from jax import lax
from jax.experimental import pallas as pl
from jax.experimental.pallas import tpu as pltpu
from jax.experimental.pallas import tpu_sc as plsc

RADIUS = 128
EMBED_DIM = 2048
T = 8192

NUM_CORES = 2
NUM_SUBCORES = 16
NUM_WORKERS = NUM_CORES * NUM_SUBCORES
BPW = T // NUM_WORKERS
ROWS = 32
NCHUNK = BPW // ROWS

_mesh = plsc.VectorSubcoreMesh(core_axis_name="c", subcore_axis_name="s")


@functools.partial(
    pl.kernel,
    mesh=_mesh,
    out_type=jax.ShapeDtypeStruct((T, EMBED_DIM), jnp.float32),
    scratch_types=[
        pltpu.VMEM((BPW,), jnp.int32),
        pltpu.VMEM((ROWS, EMBED_DIM), jnp.float32),
        pltpu.SemaphoreType.DMA,
    ],
)
def _lookup(pos_hbm, table_hbm, out_hbm, idx_v, rows_v, sem):
    wid = lax.axis_index("s") * NUM_CORES + lax.axis_index("c")
    base = wid * BPW
    pltpu.sync_copy(pos_hbm.at[pl.ds(base, BPW)], idx_v)
    for i in range(BPW // 16):
        v = idx_v[pl.ds(i * 16, 16)]
        idx_v[pl.ds(i * 16, 16)] = jnp.clip(v, -RADIUS, RADIUS) + RADIUS
    for c in range(NCHUNK):
        pltpu.async_copy(
            table_hbm.at[idx_v.at[pl.ds(c * ROWS, ROWS)]], rows_v, sem
        ).wait()
        pltpu.sync_copy(rows_v, out_hbm.at[pl.ds(base + c * ROWS, ROWS)])


def kernel(position, embed_table):
    return _lookup(position.astype(jnp.int32), embed_table)


# speedup vs baseline: 1.0795x; 1.0065x over previous
import functools

import jax
import jax.numpy as jnp
from jax import lax
from jax.experimental import pallas as pl
from jax.experimental.pallas import tpu as pltpu
from jax.experimental.pallas import tpu_sc as plsc

RADIUS = 128
EMBED_DIM = 2048
T = 8192

NUM_CORES = 2
NUM_SUBCORES = 16
NUM_WORKERS = NUM_CORES * NUM_SUBCORES
BPW = T // NUM_WORKERS
ROWS = 16
NCHUNK = BPW // ROWS

_mesh = plsc.VectorSubcoreMesh(core_axis_name="c", subcore_axis_name="s")


@functools.partial(
    pl.kernel,
    mesh=_mesh,
    out_type=jax.ShapeDtypeStruct((T, EMBED_DIM), jnp.float32),
    scratch_types=[
        pltpu.VMEM((BPW,), jnp.int32),
        pltpu.VMEM((ROWS, EMBED_DIM), jnp.float32),
        pltpu.VMEM((ROWS, EMBED_DIM), jnp.float32),
        pltpu.SemaphoreType.DMA,
        pltpu.SemaphoreType.DMA,
        pltpu.SemaphoreType.DMA,
        pltpu.SemaphoreType.DMA,
    ],
)
def _lookup(pos_hbm, table_hbm, out_hbm, idx_v, rows0, rows1, g0, g1, w0, w1):
    wid = lax.axis_index("s") * NUM_CORES + lax.axis_index("c")
    base = wid * BPW
    pltpu.sync_copy(pos_hbm.at[pl.ds(base, BPW)], idx_v)
    for i in range(BPW // 16):
        v = idx_v[pl.ds(i * 16, 16)]
        idx_v[pl.ds(i * 16, 16)] = jnp.clip(v, -RADIUS, RADIUS) + RADIUS

    bufs = (rows0, rows1)
    gsems = (g0, g1)
    wsems = (w0, w1)

    def gather(c, buf, sem):
        return pltpu.async_copy(
            table_hbm.at[idx_v.at[pl.ds(c * ROWS, ROWS)]], buf, sem
        )

    def write(c, buf, sem):
        return pltpu.async_copy(buf, out_hbm.at[pl.ds(base + c * ROWS, ROWS)], sem)

    gathers = [None] * NCHUNK
    writes = [None] * NCHUNK
    gathers[0] = gather(0, bufs[0], gsems[0])
    for c in range(NCHUNK):
        b = c % 2
        gathers[c].wait()
        if c >= 1:
            writes[c - 1].wait()
        if c + 1 < NCHUNK:
            gathers[c + 1] = gather(c + 1, bufs[1 - b], gsems[1 - b])
        writes[c] = write(c, bufs[b], wsems[b])
    writes[NCHUNK - 1].wait()


def kernel(position, embed_table):
    return _lookup(position.astype(jnp.int32), embed_table)
